# baseline (speedup 1.0000x reference)
import functools

import jax
import jax.numpy as jnp
from jax.experimental import pallas as pl

B, T, D_IN, D_CODE, K, NCB = 16, 1024, 768, 256, 1024, 2
BLK = 2048


def _body(x_ref, we_ref, be_ref, cb_ref, cbt_ref, wd_ref, bd_ref, cb0s_ref,
          c2_ref, cbb_ref, o_ref):
    x = x_ref[...]
    enc = jnp.dot(x, we_ref[...], preferred_element_type=jnp.float32) + be_ref[...]
    res = enc
    qsum = jnp.zeros_like(enc)
    for i in range(NCB):
        cb = cb_ref[i]
        cbt = cbt_ref[i]
        r2 = jnp.sum(res * res, axis=1, keepdims=True)
        c2 = c2_ref[i]
        dots = jnp.dot(res, cbt, preferred_element_type=jnp.float32)
        dist = r2 - 2.0 * dots + c2
        iota = jax.lax.broadcasted_iota(jnp.int32, dist.shape, 1)
        idx = jnp.argmin(dist, axis=1).astype(jnp.int32)
        onehot = (iota == idx[:, None]).astype(jnp.bfloat16)
        if i == 0:
            q = (jnp.dot(onehot, cb0s_ref[0], preferred_element_type=jnp.float32)
                 + jnp.dot(onehot, cb0s_ref[1], preferred_element_type=jnp.float32)
                 + jnp.dot(onehot, cb0s_ref[2], preferred_element_type=jnp.float32))
        else:
            q = jnp.dot(onehot, cbb_ref[...], preferred_element_type=jnp.float32)
        qsum = qsum + q
        res = res - q
    codes = enc + (qsum - enc)
    o_ref[...] = jnp.dot(codes, wd_ref[...], preferred_element_type=jnp.float32) + bd_ref[...]


@jax.jit
def _run(xf, W_enc, b_enc, W_dec, b_dec, codebooks, cbT, cb0_split, c2_all,
         cb1_bf16):
    n_blk = (B * T) // BLK
    return pl.pallas_call(
        _body,
        grid=(n_blk,),
        in_specs=[
            pl.BlockSpec((BLK, D_IN), lambda i: (i, 0)),
            pl.BlockSpec((D_IN, D_CODE), lambda i: (0, 0)),
            pl.BlockSpec((1, D_CODE), lambda i: (0, 0)),
            pl.BlockSpec((NCB, K, D_CODE), lambda i: (0, 0, 0)),
            pl.BlockSpec((NCB, D_CODE, K), lambda i: (0, 0, 0)),
            pl.BlockSpec((D_CODE, D_IN), lambda i: (0, 0)),
            pl.BlockSpec((1, D_IN), lambda i: (0, 0)),
            pl.BlockSpec((3, K, D_CODE), lambda i: (0, 0, 0)),
            pl.BlockSpec((NCB, 1, K), lambda i: (0, 0, 0)),
            pl.BlockSpec((K, D_CODE), lambda i: (0, 0)),
        ],
        out_specs=pl.BlockSpec((BLK, D_IN), lambda i: (i, 0)),
        out_shape=jax.ShapeDtypeStruct((B * T, D_IN), jnp.float32),
    )(xf, W_enc, b_enc.reshape(1, D_CODE), codebooks, cbT, W_dec,
      b_dec.reshape(1, D_IN), cb0_split, c2_all, cb1_bf16)


def _split3(a):
    def trunc(v):
        bits = jax.lax.bitcast_convert_type(v, jnp.uint32)
        return jax.lax.bitcast_convert_type(bits & jnp.uint32(0xFFFF0000),
                                            jnp.float32)
    hi = trunc(a)
    r1 = a - hi
    mid = trunc(r1)
    r2 = r1 - mid
    lo = trunc(r2)
    return jnp.stack([hi, mid, lo]).astype(jnp.bfloat16)


def kernel(x, W_enc, b_enc, W_dec, b_dec, codebooks):
    xf = x.reshape(B * T, D_IN)
    cbT = jnp.transpose(codebooks, (0, 2, 1))
    cb0_split = _split3(codebooks[0])
    c2_all = jnp.stack([jnp.sum(codebooks[i] ** 2, axis=-1)[None, :]
                        for i in range(NCB)])
    cb1_bf16 = codebooks[1].astype(jnp.bfloat16)
    out = _run(xf, W_enc, b_enc, W_dec, b_dec, codebooks, cbT, cb0_split,
               c2_all, cb1_bf16)
    return out.reshape(B, T, D_IN)


# speedup vs baseline: 1.1128x; 1.1128x over previous
import functools

import jax
import jax.numpy as jnp
from jax.experimental import pallas as pl

B, T, D_IN, D_CODE, K, NCB = 16, 1024, 768, 256, 1024, 2
BLK = 2048


def _body(x_ref, we_ref, be_ref, cb_ref, cbt_ref, wd_ref, bd_ref, cb0s_ref,
          c2_ref, cbb_ref, o_ref):
    x = x_ref[...]
    enc = jnp.dot(x, we_ref[...], preferred_element_type=jnp.float32) + be_ref[...]
    res = enc
    qsum = jnp.zeros_like(enc)
    for i in range(NCB):
        cb = cb_ref[i]
        cbt = cbt_ref[i]
        r2 = jnp.sum(res * res, axis=1, keepdims=True)
        c2 = c2_ref[i]
        dots = jnp.dot(res, cbt, preferred_element_type=jnp.float32)
        dist = r2 - 2.0 * dots + c2
        iota = jax.lax.broadcasted_iota(jnp.int32, dist.shape, 1)
        idx = jnp.argmin(dist, axis=1).astype(jnp.int32)
        onehot = (iota == idx[:, None]).astype(jnp.float32)
        if i == 0:
            q = (jnp.dot(onehot, cb0s_ref[0], preferred_element_type=jnp.float32)
                 + jnp.dot(onehot, cb0s_ref[1], preferred_element_type=jnp.float32)
                 + jnp.dot(onehot, cb0s_ref[2], preferred_element_type=jnp.float32))
        else:
            q = jnp.dot(onehot, cb, preferred_element_type=jnp.float32)
        qsum = qsum + q
        res = res - q
    codes = enc + (qsum - enc)
    o_ref[...] = jnp.dot(codes, wd_ref[...], preferred_element_type=jnp.float32) + bd_ref[...]


@jax.jit
def _run(xf, W_enc, b_enc, W_dec, b_dec, codebooks, cbT, cb0_split, c2_all,
         cb1_bf16):
    n_blk = (B * T) // BLK
    return pl.pallas_call(
        _body,
        grid=(n_blk,),
        in_specs=[
            pl.BlockSpec((BLK, D_IN), lambda i: (i, 0)),
            pl.BlockSpec((D_IN, D_CODE), lambda i: (0, 0)),
            pl.BlockSpec((1, D_CODE), lambda i: (0, 0)),
            pl.BlockSpec((NCB, K, D_CODE), lambda i: (0, 0, 0)),
            pl.BlockSpec((NCB, D_CODE, K), lambda i: (0, 0, 0)),
            pl.BlockSpec((D_CODE, D_IN), lambda i: (0, 0)),
            pl.BlockSpec((1, D_IN), lambda i: (0, 0)),
            pl.BlockSpec((3, K, D_CODE), lambda i: (0, 0, 0)),
            pl.BlockSpec((NCB, 1, K), lambda i: (0, 0, 0)),
            pl.BlockSpec((K, D_CODE), lambda i: (0, 0)),
        ],
        out_specs=pl.BlockSpec((BLK, D_IN), lambda i: (i, 0)),
        out_shape=jax.ShapeDtypeStruct((B * T, D_IN), jnp.float32),
    )(xf, W_enc, b_enc.reshape(1, D_CODE), codebooks, cbT, W_dec,
      b_dec.reshape(1, D_IN), cb0_split, c2_all, cb1_bf16)


def _split3(a):
    def trunc(v):
        bits = jax.lax.bitcast_convert_type(v, jnp.uint32)
        return jax.lax.bitcast_convert_type(bits & jnp.uint32(0xFFFF0000),
                                            jnp.float32)
    hi = trunc(a)
    r1 = a - hi
    mid = trunc(r1)
    r2 = r1 - mid
    lo = trunc(r2)
    return jnp.stack([hi, mid, lo])


def kernel(x, W_enc, b_enc, W_dec, b_dec, codebooks):
    xf = x.reshape(B * T, D_IN)
    cbT = jnp.transpose(codebooks, (0, 2, 1))
    cb0_split = _split3(codebooks[0])
    c2_all = jnp.stack([jnp.sum(codebooks[i] ** 2, axis=-1)[None, :]
                        for i in range(NCB)])
    cb1_bf16 = codebooks[1].astype(jnp.bfloat16)
    out = _run(xf, W_enc, b_enc, W_dec, b_dec, codebooks, cbT, cb0_split,
               c2_all, cb1_bf16)
    return out.reshape(B, T, D_IN)
